# initial kernel scaffold (unmeasured)
import jax
import jax.numpy as jnp
from jax import lax
from jax.experimental import pallas as pl
from jax.experimental.pallas import tpu as pltpu

N_DEV = 8
M, K_SHARD, N = 4096, 512, 2048
BLK = M // N_DEV


def kernel(x, w_mat):
    def body(x_ref, w_ref, out_ref, comm_ref, pc_ref, amax_ref,
             rs_send, rs_recv, am_send, am_recv, ag_send, ag_recv):
        my = lax.axis_index("i")
        left = lax.rem(my + N_DEV - 1, N_DEV)
        right = lax.rem(my + 1, N_DEV)

        def rows(c):
            return pl.ds(c * BLK, BLK)

        def pc(c):
            return jnp.dot(x_ref[rows(c), :], w_ref[...],
                           preferred_element_type=jnp.float32)

        def c_of(s):
            return lax.rem(my - s + N_DEV, N_DEV)

        comm_ref[0] = pc(c_of(0))

        barrier_sem = pltpu.get_barrier_semaphore()
        for nbr in [left, right]:
            pl.semaphore_signal(barrier_sem, inc=1, device_id=(nbr,),
                                device_id_type=pl.DeviceIdType.MESH)
        pl.semaphore_wait(barrier_sem, 2)

        for s in range(N_DEV - 1):
            send_slot, recv_slot = s % 2, (s + 1) % 2
            rdma = pltpu.make_async_remote_copy(
                src_ref=comm_ref.at[send_slot],
                dst_ref=comm_ref.at[recv_slot],
                send_sem=rs_send.at[send_slot],
                recv_sem=rs_recv.at[recv_slot],
                device_id=(right,),
                device_id_type=pl.DeviceIdType.MESH,
            )
            rdma.start()
            pc_ref[...] = pc(c_of(s + 1))
            rdma.wait()
            comm_ref[recv_slot] = comm_ref[recv_slot] + pc_ref[...]

        mine = c_of(N_DEV - 1)
        pc_ref[...] = jnp.maximum(comm_ref[(N_DEV - 1) % 2], 0.0)

        amax_ref[0] = jnp.zeros((8, 128), jnp.float32) + jnp.max(pc_ref[...])
        for s in range(N_DEV - 1):
            send_slot, recv_slot = s % 2, (s + 1) % 2
            rdma = pltpu.make_async_remote_copy(
                src_ref=amax_ref.at[send_slot],
                dst_ref=amax_ref.at[recv_slot],
                send_sem=am_send.at[send_slot],
                recv_sem=am_recv.at[recv_slot],
                device_id=(right,),
                device_id_type=pl.DeviceIdType.MESH,
            )
            rdma.start()
            rdma.wait()
            amax_ref[recv_slot] = jnp.maximum(amax_ref[recv_slot],
                                              amax_ref[send_slot])

        gmax = amax_ref[(N_DEV - 1) % 2, 0, 0]

        scale = gmax / 127.0
        inv = jnp.where(gmax > 0.0, 127.0 / gmax, 0.0)
        q = jnp.clip(jnp.round(pc_ref[...] * inv), -127.0, 127.0)
        out_ref[rows(mine), :] = q * scale

        for s in range(N_DEV - 1):
            slot = s % 2
            send_c = lax.rem(my + 1 - s + N_DEV, N_DEV)
            rdma = pltpu.make_async_remote_copy(
                src_ref=out_ref.at[rows(send_c)],
                dst_ref=out_ref.at[rows(send_c)],
                send_sem=ag_send.at[slot],
                recv_sem=ag_recv.at[slot],
                device_id=(right,),
                device_id_type=pl.DeviceIdType.MESH,
            )
            rdma.start()
            rdma.wait()

    return pl.pallas_call(
        body,
        out_shape=jax.ShapeDtypeStruct((M, N), jnp.float32),
        in_specs=[pl.BlockSpec(memory_space=pltpu.VMEM),
                  pl.BlockSpec(memory_space=pltpu.VMEM)],
        out_specs=pl.BlockSpec(memory_space=pltpu.VMEM),
        scratch_shapes=[
            pltpu.VMEM((2, BLK, N), jnp.float32),
            pltpu.VMEM((BLK, N), jnp.float32),
            pltpu.VMEM((2, 8, 128), jnp.float32),
            pltpu.SemaphoreType.DMA((2,)),
            pltpu.SemaphoreType.DMA((2,)),
            pltpu.SemaphoreType.DMA((2,)),
            pltpu.SemaphoreType.DMA((2,)),
            pltpu.SemaphoreType.DMA((2,)),
            pltpu.SemaphoreType.DMA((2,)),
        ],
        compiler_params=pltpu.CompilerParams(collective_id=0),
    )(x, w_mat)


# baseline (device time: 716976 ns/iter reference)
import jax
import jax.numpy as jnp
from jax import lax
from jax.experimental import pallas as pl
from jax.experimental.pallas import tpu as pltpu

N_DEV = 8
M, K_SHARD, N = 4096, 512, 2048
BLK = M // N_DEV


def kernel(x, w_mat):
    def body(x_ref, w_ref, out_ref, comm_ref, pc_ref, amax_ref,
             rs_send, rs_recv, am_send, am_recv, ag_send, ag_recv):
        my = lax.axis_index("i")
        left = lax.rem(my + N_DEV - 1, N_DEV)
        right = lax.rem(my + 1, N_DEV)

        def rows(c):
            return pl.ds(c * BLK, BLK)

        def pc(c):
            return jnp.dot(x_ref[rows(c), :], w_ref[...],
                           preferred_element_type=jnp.float32)

        def c_of(s):
            return lax.rem(my - s + N_DEV, N_DEV)

        comm_ref[0] = pc(c_of(0))

        barrier_sem = pltpu.get_barrier_semaphore()
        for nbr in [left, right]:
            pl.semaphore_signal(barrier_sem, inc=1, device_id=(nbr,),
                                device_id_type=pl.DeviceIdType.MESH)
        pl.semaphore_wait(barrier_sem, 2)

        for s in range(N_DEV - 1):
            send_slot, recv_slot = s % 2, (s + 1) % 2
            rdma = pltpu.make_async_remote_copy(
                src_ref=comm_ref.at[send_slot],
                dst_ref=comm_ref.at[recv_slot],
                send_sem=rs_send.at[send_slot],
                recv_sem=rs_recv.at[recv_slot],
                device_id=(right,),
                device_id_type=pl.DeviceIdType.MESH,
            )
            rdma.start()
            pc_ref[...] = pc(c_of(s + 1))
            rdma.wait()
            comm_ref[recv_slot] = comm_ref[recv_slot] + pc_ref[...]

        mine = c_of(N_DEV - 1)
        pc_ref[...] = jnp.maximum(comm_ref[(N_DEV - 1) % 2], 0.0)

        amax_ref[0] = jnp.zeros((8, 128), jnp.float32) + jnp.max(pc_ref[...])
        for s in range(N_DEV - 1):
            send_slot, recv_slot = s % 2, (s + 1) % 2
            rdma = pltpu.make_async_remote_copy(
                src_ref=amax_ref.at[send_slot],
                dst_ref=amax_ref.at[recv_slot],
                send_sem=am_send.at[send_slot],
                recv_sem=am_recv.at[recv_slot],
                device_id=(right,),
                device_id_type=pl.DeviceIdType.MESH,
            )
            rdma.start()
            rdma.wait()
            amax_ref[recv_slot] = jnp.maximum(amax_ref[recv_slot],
                                              amax_ref[send_slot])

        gmax = amax_ref[(N_DEV - 1) % 2, 0, 0]

        scale = gmax / 127.0
        inv = jnp.where(gmax > 0.0, 127.0 / gmax, 0.0)
        q = jnp.clip(jnp.round(pc_ref[...] * inv), -127.0, 127.0)
        out_ref[rows(mine), :] = q * scale

        for s in range(N_DEV - 1):
            slot = s % 2
            send_c = lax.rem(my + 1 - s + N_DEV, N_DEV)
            rdma = pltpu.make_async_remote_copy(
                src_ref=out_ref.at[rows(send_c)],
                dst_ref=out_ref.at[rows(send_c)],
                send_sem=ag_send.at[slot],
                recv_sem=ag_recv.at[slot],
                device_id=(right,),
                device_id_type=pl.DeviceIdType.MESH,
            )
            rdma.start()
            rdma.wait()

    return pl.pallas_call(
        body,
        out_shape=jax.ShapeDtypeStruct((M, N), jnp.float32),
        in_specs=[pl.BlockSpec(memory_space=pltpu.VMEM),
                  pl.BlockSpec(memory_space=pltpu.VMEM)],
        out_specs=pl.BlockSpec(memory_space=pltpu.VMEM),
        scratch_shapes=[
            pltpu.VMEM((2, BLK, N), jnp.float32),
            pltpu.VMEM((BLK, N), jnp.float32),
            pltpu.VMEM((2, 8, 128), jnp.float32),
            pltpu.SemaphoreType.DMA((2,)),
            pltpu.SemaphoreType.DMA((2,)),
            pltpu.SemaphoreType.DMA((2,)),
            pltpu.SemaphoreType.DMA((2,)),
            pltpu.SemaphoreType.DMA((2,)),
            pltpu.SemaphoreType.DMA((2,)),
        ],
        compiler_params=pltpu.CompilerParams(
            collective_id=0, vmem_limit_bytes=60 * 1024 * 1024),
    )(x, w_mat)


# device time: 485244 ns/iter; 1.4776x vs baseline; 1.4776x over previous
import jax
import jax.numpy as jnp
from jax import lax
from jax.experimental import pallas as pl
from jax.experimental.pallas import tpu as pltpu

N_DEV = 8
M, K_SHARD, N = 4096, 512, 2048
BLK = M // N_DEV


def kernel(x, w_mat):
    def body(x_ref, w_ref, out_ref, comm_ref, pc_ref, amax_ref, q_ref,
             rs_send, rs_recv, am_send, am_recv, ag_send, ag_recv):
        my = lax.axis_index("i")
        left = lax.rem(my + N_DEV - 1, N_DEV)
        right = lax.rem(my + 1, N_DEV)

        def rows(c):
            return pl.ds(c * BLK, BLK)

        def pc(c):
            return jnp.dot(x_ref[rows(c), :], w_ref[...],
                           preferred_element_type=jnp.float32)

        def c_of(s):
            return lax.rem(my - s + N_DEV, N_DEV)

        comm_ref[0] = pc(c_of(0))

        barrier_sem = pltpu.get_barrier_semaphore()
        for nbr in [left, right]:
            pl.semaphore_signal(barrier_sem, inc=1, device_id=(nbr,),
                                device_id_type=pl.DeviceIdType.MESH)
        pl.semaphore_wait(barrier_sem, 2)

        for s in range(N_DEV - 1):
            send_slot, recv_slot = s % 2, (s + 1) % 2
            rdma = pltpu.make_async_remote_copy(
                src_ref=comm_ref.at[send_slot],
                dst_ref=comm_ref.at[recv_slot],
                send_sem=rs_send.at[send_slot],
                recv_sem=rs_recv.at[recv_slot],
                device_id=(right,),
                device_id_type=pl.DeviceIdType.MESH,
            )
            rdma.start()
            pc_ref[...] = pc(c_of(s + 1))
            rdma.wait()
            comm_ref[recv_slot] = comm_ref[recv_slot] + pc_ref[...]

        mine = c_of(N_DEV - 1)
        pc_ref[...] = jnp.maximum(comm_ref[(N_DEV - 1) % 2], 0.0)

        amax_ref[0] = jnp.zeros((8, 128), jnp.float32) + jnp.max(pc_ref[...])
        for s in range(N_DEV - 1):
            send_slot, recv_slot = s % 2, (s + 1) % 2
            rdma = pltpu.make_async_remote_copy(
                src_ref=amax_ref.at[send_slot],
                dst_ref=amax_ref.at[recv_slot],
                send_sem=am_send.at[send_slot],
                recv_sem=am_recv.at[recv_slot],
                device_id=(right,),
                device_id_type=pl.DeviceIdType.MESH,
            )
            rdma.start()
            rdma.wait()
            amax_ref[recv_slot] = jnp.maximum(amax_ref[recv_slot],
                                              amax_ref[send_slot])

        gmax = amax_ref[(N_DEV - 1) % 2, 0, 0]

        scale = gmax / 127.0
        inv = jnp.where(gmax > 0.0, 127.0 / gmax, 0.0)
        q = jnp.clip(jnp.round(pc_ref[...] * inv), -127.0, 127.0)
        q_ref[0] = q.astype(jnp.int8)
        out_ref[rows(mine), :] = q_ref[0].astype(jnp.float32) * scale

        for s in range(N_DEV - 1):
            send_slot, recv_slot = s % 2, (s + 1) % 2
            rdma = pltpu.make_async_remote_copy(
                src_ref=q_ref.at[send_slot],
                dst_ref=q_ref.at[recv_slot],
                send_sem=ag_send.at[send_slot],
                recv_sem=ag_recv.at[recv_slot],
                device_id=(right,),
                device_id_type=pl.DeviceIdType.MESH,
            )
            rdma.start()
            rdma.wait()
            recv_c = lax.rem(my - s + N_DEV, N_DEV)
            out_ref[rows(recv_c), :] = q_ref[recv_slot].astype(jnp.float32) * scale

    return pl.pallas_call(
        body,
        out_shape=jax.ShapeDtypeStruct((M, N), jnp.float32),
        in_specs=[pl.BlockSpec(memory_space=pltpu.VMEM),
                  pl.BlockSpec(memory_space=pltpu.VMEM)],
        out_specs=pl.BlockSpec(memory_space=pltpu.VMEM),
        scratch_shapes=[
            pltpu.VMEM((2, BLK, N), jnp.float32),
            pltpu.VMEM((BLK, N), jnp.float32),
            pltpu.VMEM((2, 8, 128), jnp.float32),
            pltpu.VMEM((2, BLK, N), jnp.int8),
            pltpu.SemaphoreType.DMA((2,)),
            pltpu.SemaphoreType.DMA((2,)),
            pltpu.SemaphoreType.DMA((2,)),
            pltpu.SemaphoreType.DMA((2,)),
            pltpu.SemaphoreType.DMA((2,)),
            pltpu.SemaphoreType.DMA((2,)),
        ],
        compiler_params=pltpu.CompilerParams(
            collective_id=0, vmem_limit_bytes=60 * 1024 * 1024),
    )(x, w_mat)


# device time: 291328 ns/iter; 2.4611x vs baseline; 1.6656x over previous
import jax
import jax.numpy as jnp
from jax import lax
from jax.experimental import pallas as pl
from jax.experimental.pallas import tpu as pltpu

N_DEV = 8
M, K_SHARD, N = 4096, 512, 2048
BLK = M // N_DEV
HN = N // 2


def kernel(x, w_mat):
    def body(x_ref, w_ref, out_ref, commR_ref, commL_ref, pc_ref, amax_ref,
             qR_ref, qL_ref, rsR_send, rsR_recv, rsL_send, rsL_recv,
             am_send, am_recv, agR_send, agR_recv, agL_send, agL_recv):
        my = lax.axis_index("i")
        left = lax.rem(my + N_DEV - 1, N_DEV)
        right = lax.rem(my + 1, N_DEV)

        def rows(c):
            return pl.ds(c * BLK, BLK)

        def pcR(c):
            return jnp.dot(x_ref[rows(c), :], w_ref[:, :HN],
                           preferred_element_type=jnp.float32)

        def pcL(c):
            return jnp.dot(x_ref[rows(c), :], w_ref[:, HN:],
                           preferred_element_type=jnp.float32)

        def cR(s):
            return lax.rem(my - s + 2 * N_DEV, N_DEV)

        def cL(s):
            return lax.rem(my + s, N_DEV)

        commR_ref[0] = pcR(cR(0))
        commL_ref[0] = pcL(cL(0))

        barrier_sem = pltpu.get_barrier_semaphore()
        for nbr in [left, right]:
            pl.semaphore_signal(barrier_sem, inc=1, device_id=(nbr,),
                                device_id_type=pl.DeviceIdType.MESH)
        pl.semaphore_wait(barrier_sem, 2)

        for s in range(N_DEV - 1):
            send_slot, recv_slot = s % 2, (s + 1) % 2
            rdmaR = pltpu.make_async_remote_copy(
                src_ref=commR_ref.at[send_slot],
                dst_ref=commR_ref.at[recv_slot],
                send_sem=rsR_send.at[send_slot],
                recv_sem=rsR_recv.at[recv_slot],
                device_id=(right,),
                device_id_type=pl.DeviceIdType.MESH,
            )
            rdmaL = pltpu.make_async_remote_copy(
                src_ref=commL_ref.at[send_slot],
                dst_ref=commL_ref.at[recv_slot],
                send_sem=rsL_send.at[send_slot],
                recv_sem=rsL_recv.at[recv_slot],
                device_id=(left,),
                device_id_type=pl.DeviceIdType.MESH,
            )
            rdmaR.start()
            rdmaL.start()
            pc_ref[:, :HN] = pcR(cR(s + 1))
            pc_ref[:, HN:] = pcL(cL(s + 1))
            rdmaR.wait()
            commR_ref[recv_slot] = commR_ref[recv_slot] + pc_ref[:, :HN]
            rdmaL.wait()
            commL_ref[recv_slot] = commL_ref[recv_slot] + pc_ref[:, HN:]

        mineR = cR(N_DEV - 1)
        mineL = cL(N_DEV - 1)
        pc_ref[:, :HN] = jnp.maximum(commR_ref[(N_DEV - 1) % 2], 0.0)
        pc_ref[:, HN:] = jnp.maximum(commL_ref[(N_DEV - 1) % 2], 0.0)

        amax_ref[0] = jnp.zeros((8, 128), jnp.float32) + jnp.max(pc_ref[...])
        for s in range(N_DEV - 1):
            send_slot, recv_slot = s % 2, (s + 1) % 2
            rdma = pltpu.make_async_remote_copy(
                src_ref=amax_ref.at[send_slot],
                dst_ref=amax_ref.at[recv_slot],
                send_sem=am_send.at[send_slot],
                recv_sem=am_recv.at[recv_slot],
                device_id=(right,),
                device_id_type=pl.DeviceIdType.MESH,
            )
            rdma.start()
            rdma.wait()
            amax_ref[recv_slot] = jnp.maximum(amax_ref[recv_slot],
                                              amax_ref[send_slot])

        gmax = amax_ref[(N_DEV - 1) % 2, 0, 0]

        scale = gmax / 127.0
        inv = jnp.where(gmax > 0.0, 127.0 / gmax, 0.0)
        qR_ref[0] = jnp.clip(jnp.round(pc_ref[:, :HN] * inv),
                             -127.0, 127.0).astype(jnp.int8)
        qL_ref[0] = jnp.clip(jnp.round(pc_ref[:, HN:] * inv),
                             -127.0, 127.0).astype(jnp.int8)
        out_ref[rows(mineR), :HN] = qR_ref[0].astype(jnp.float32) * scale
        out_ref[rows(mineL), HN:] = qL_ref[0].astype(jnp.float32) * scale

        for s in range(N_DEV - 1):
            send_slot, recv_slot = s % 2, (s + 1) % 2
            rdmaR = pltpu.make_async_remote_copy(
                src_ref=qR_ref.at[send_slot],
                dst_ref=qR_ref.at[recv_slot],
                send_sem=agR_send.at[send_slot],
                recv_sem=agR_recv.at[recv_slot],
                device_id=(right,),
                device_id_type=pl.DeviceIdType.MESH,
            )
            rdmaL = pltpu.make_async_remote_copy(
                src_ref=qL_ref.at[send_slot],
                dst_ref=qL_ref.at[recv_slot],
                send_sem=agL_send.at[send_slot],
                recv_sem=agL_recv.at[recv_slot],
                device_id=(left,),
                device_id_type=pl.DeviceIdType.MESH,
            )
            rdmaR.start()
            rdmaL.start()
            recv_cR = lax.rem(my - s + 2 * N_DEV, N_DEV)
            recv_cL = lax.rem(my + s, N_DEV)
            rdmaR.wait()
            out_ref[rows(recv_cR), :HN] = (
                qR_ref[recv_slot].astype(jnp.float32) * scale)
            rdmaL.wait()
            out_ref[rows(recv_cL), HN:] = (
                qL_ref[recv_slot].astype(jnp.float32) * scale)

    return pl.pallas_call(
        body,
        out_shape=jax.ShapeDtypeStruct((M, N), jnp.float32),
        in_specs=[pl.BlockSpec(memory_space=pltpu.VMEM),
                  pl.BlockSpec(memory_space=pltpu.VMEM)],
        out_specs=pl.BlockSpec(memory_space=pltpu.VMEM),
        scratch_shapes=[
            pltpu.VMEM((2, BLK, HN), jnp.float32),
            pltpu.VMEM((2, BLK, HN), jnp.float32),
            pltpu.VMEM((BLK, N), jnp.float32),
            pltpu.VMEM((2, 8, 128), jnp.float32),
            pltpu.VMEM((2, BLK, HN), jnp.int8),
            pltpu.VMEM((2, BLK, HN), jnp.int8),
            pltpu.SemaphoreType.DMA((2,)),
            pltpu.SemaphoreType.DMA((2,)),
            pltpu.SemaphoreType.DMA((2,)),
            pltpu.SemaphoreType.DMA((2,)),
            pltpu.SemaphoreType.DMA((2,)),
            pltpu.SemaphoreType.DMA((2,)),
            pltpu.SemaphoreType.DMA((2,)),
            pltpu.SemaphoreType.DMA((2,)),
            pltpu.SemaphoreType.DMA((2,)),
            pltpu.SemaphoreType.DMA((2,)),
        ],
        compiler_params=pltpu.CompilerParams(
            collective_id=0, vmem_limit_bytes=60 * 1024 * 1024),
    )(x, w_mat)


# device time: 214191 ns/iter; 3.3474x vs baseline; 1.3601x over previous
import jax
import jax.numpy as jnp
from jax import lax
from jax.experimental import pallas as pl
from jax.experimental.pallas import tpu as pltpu

N_DEV = 8
M, K_SHARD, N = 4096, 512, 2048
BLK = M // N_DEV
HN = N // 2


def kernel(x, w_mat):
    def body(x_ref, w_ref, out_ref, commR_ref, commL_ref, pc_ref, amax_ref,
             qR_ref, qL_ref, rsR_send, rsR_recv, rsL_send, rsL_recv,
             am_send, am_recv, agR_send, agR_recv, agL_send, agL_recv):
        my = lax.axis_index("i")
        left = lax.rem(my + N_DEV - 1, N_DEV)
        right = lax.rem(my + 1, N_DEV)

        def rows(c):
            return pl.ds(c * BLK, BLK)

        def pcR(c):
            return jnp.dot(x_ref[rows(c), :], w_ref[:, :HN],
                           preferred_element_type=jnp.float32)

        def pcL(c):
            return jnp.dot(x_ref[rows(c), :], w_ref[:, HN:],
                           preferred_element_type=jnp.float32)

        def cR(s):
            return lax.rem(my - s + 2 * N_DEV, N_DEV)

        def cL(s):
            return lax.rem(my + s, N_DEV)

        commR_ref[0] = pcR(cR(0)).astype(jnp.bfloat16)
        commL_ref[0] = pcL(cL(0)).astype(jnp.bfloat16)

        barrier_sem = pltpu.get_barrier_semaphore()
        for nbr in [left, right]:
            pl.semaphore_signal(barrier_sem, inc=1, device_id=(nbr,),
                                device_id_type=pl.DeviceIdType.MESH)
        pl.semaphore_wait(barrier_sem, 2)

        for s in range(N_DEV - 1):
            send_slot, recv_slot = s % 2, (s + 1) % 2
            rdmaR = pltpu.make_async_remote_copy(
                src_ref=commR_ref.at[send_slot],
                dst_ref=commR_ref.at[recv_slot],
                send_sem=rsR_send.at[send_slot],
                recv_sem=rsR_recv.at[recv_slot],
                device_id=(right,),
                device_id_type=pl.DeviceIdType.MESH,
            )
            rdmaL = pltpu.make_async_remote_copy(
                src_ref=commL_ref.at[send_slot],
                dst_ref=commL_ref.at[recv_slot],
                send_sem=rsL_send.at[send_slot],
                recv_sem=rsL_recv.at[recv_slot],
                device_id=(left,),
                device_id_type=pl.DeviceIdType.MESH,
            )
            rdmaR.start()
            rdmaL.start()
            pc_ref[:, :HN] = pcR(cR(s + 1))
            pc_ref[:, HN:] = pcL(cL(s + 1))
            rdmaR.wait()
            sumR = (commR_ref[recv_slot].astype(jnp.float32)
                    + pc_ref[:, :HN])
            rdmaL.wait()
            sumL = (commL_ref[recv_slot].astype(jnp.float32)
                    + pc_ref[:, HN:])
            if s < N_DEV - 2:
                commR_ref[recv_slot] = sumR.astype(jnp.bfloat16)
                commL_ref[recv_slot] = sumL.astype(jnp.bfloat16)
            else:
                pc_ref[:, :HN] = jnp.maximum(sumR, 0.0)
                pc_ref[:, HN:] = jnp.maximum(sumL, 0.0)

        mineR = cR(N_DEV - 1)
        mineL = cL(N_DEV - 1)

        amax_ref[0] = jnp.zeros((8, 128), jnp.float32) + jnp.max(pc_ref[...])
        for s in range(N_DEV - 1):
            send_slot, recv_slot = s % 2, (s + 1) % 2
            rdma = pltpu.make_async_remote_copy(
                src_ref=amax_ref.at[send_slot],
                dst_ref=amax_ref.at[recv_slot],
                send_sem=am_send.at[send_slot],
                recv_sem=am_recv.at[recv_slot],
                device_id=(right,),
                device_id_type=pl.DeviceIdType.MESH,
            )
            rdma.start()
            rdma.wait()
            amax_ref[recv_slot] = jnp.maximum(amax_ref[recv_slot],
                                              amax_ref[send_slot])

        gmax = amax_ref[(N_DEV - 1) % 2, 0, 0]

        scale = gmax / 127.0
        inv = jnp.where(gmax > 0.0, 127.0 / gmax, 0.0)
        qR_ref[0] = jnp.clip(jnp.round(pc_ref[:, :HN] * inv),
                             -127.0, 127.0).astype(jnp.int8)
        qL_ref[0] = jnp.clip(jnp.round(pc_ref[:, HN:] * inv),
                             -127.0, 127.0).astype(jnp.int8)
        out_ref[rows(mineR), :HN] = qR_ref[0].astype(jnp.float32) * scale
        out_ref[rows(mineL), HN:] = qL_ref[0].astype(jnp.float32) * scale

        for s in range(N_DEV - 1):
            send_slot, recv_slot = s % 2, (s + 1) % 2
            rdmaR = pltpu.make_async_remote_copy(
                src_ref=qR_ref.at[send_slot],
                dst_ref=qR_ref.at[recv_slot],
                send_sem=agR_send.at[send_slot],
                recv_sem=agR_recv.at[recv_slot],
                device_id=(right,),
                device_id_type=pl.DeviceIdType.MESH,
            )
            rdmaL = pltpu.make_async_remote_copy(
                src_ref=qL_ref.at[send_slot],
                dst_ref=qL_ref.at[recv_slot],
                send_sem=agL_send.at[send_slot],
                recv_sem=agL_recv.at[recv_slot],
                device_id=(left,),
                device_id_type=pl.DeviceIdType.MESH,
            )
            rdmaR.start()
            rdmaL.start()
            recv_cR = lax.rem(my - s + 2 * N_DEV, N_DEV)
            recv_cL = lax.rem(my + s, N_DEV)
            rdmaR.wait()
            out_ref[rows(recv_cR), :HN] = (
                qR_ref[recv_slot].astype(jnp.float32) * scale)
            rdmaL.wait()
            out_ref[rows(recv_cL), HN:] = (
                qL_ref[recv_slot].astype(jnp.float32) * scale)

    return pl.pallas_call(
        body,
        out_shape=jax.ShapeDtypeStruct((M, N), jnp.float32),
        in_specs=[pl.BlockSpec(memory_space=pltpu.VMEM),
                  pl.BlockSpec(memory_space=pltpu.VMEM)],
        out_specs=pl.BlockSpec(memory_space=pltpu.VMEM),
        scratch_shapes=[
            pltpu.VMEM((2, BLK, HN), jnp.bfloat16),
            pltpu.VMEM((2, BLK, HN), jnp.bfloat16),
            pltpu.VMEM((BLK, N), jnp.float32),
            pltpu.VMEM((2, 8, 128), jnp.float32),
            pltpu.VMEM((2, BLK, HN), jnp.int8),
            pltpu.VMEM((2, BLK, HN), jnp.int8),
            pltpu.SemaphoreType.DMA((2,)),
            pltpu.SemaphoreType.DMA((2,)),
            pltpu.SemaphoreType.DMA((2,)),
            pltpu.SemaphoreType.DMA((2,)),
            pltpu.SemaphoreType.DMA((2,)),
            pltpu.SemaphoreType.DMA((2,)),
            pltpu.SemaphoreType.DMA((2,)),
            pltpu.SemaphoreType.DMA((2,)),
            pltpu.SemaphoreType.DMA((2,)),
            pltpu.SemaphoreType.DMA((2,)),
        ],
        compiler_params=pltpu.CompilerParams(
            collective_id=0, vmem_limit_bytes=60 * 1024 * 1024),
    )(x, w_mat)


# device time: 202242 ns/iter; 3.5451x vs baseline; 1.0591x over previous
import jax
import jax.numpy as jnp
from jax import lax
from jax.experimental import pallas as pl
from jax.experimental.pallas import tpu as pltpu

N_DEV = 8
M, K_SHARD, N = 4096, 512, 2048
BLK = M // N_DEV
HN = N // 2


def kernel(x, w_mat):
    def body(x_ref, w_ref, out_ref, commR_ref, commL_ref, pc_ref, amax_ref,
             qR_ref, qL_ref, rsR_send, rsR_recv, rsL_send, rsL_recv,
             am_send, am_recv, agR_send, agR_recv, agL_send, agL_recv):
        my = lax.axis_index("i")
        left = lax.rem(my + N_DEV - 1, N_DEV)
        right = lax.rem(my + 1, N_DEV)

        def rows(c):
            return pl.ds(c * BLK, BLK)

        def pcR(c):
            return jnp.dot(x_ref[rows(c), :], w_ref[:, :HN],
                           preferred_element_type=jnp.float32)

        def pcL(c):
            return jnp.dot(x_ref[rows(c), :], w_ref[:, HN:],
                           preferred_element_type=jnp.float32)

        def cR(s):
            return lax.rem(my - s + 2 * N_DEV, N_DEV)

        def cL(s):
            return lax.rem(my + s, N_DEV)

        commR_ref[0] = pcR(cR(0)).astype(jnp.bfloat16)
        commL_ref[0] = pcL(cL(0)).astype(jnp.bfloat16)

        barrier_sem = pltpu.get_barrier_semaphore()
        for nbr in [left, right]:
            pl.semaphore_signal(barrier_sem, inc=1, device_id=(nbr,),
                                device_id_type=pl.DeviceIdType.MESH)
        pl.semaphore_wait(barrier_sem, 2)

        for s in range(N_DEV - 1):
            send_slot, recv_slot = s % 2, (s + 1) % 2
            rdmaR = pltpu.make_async_remote_copy(
                src_ref=commR_ref.at[send_slot],
                dst_ref=commR_ref.at[recv_slot],
                send_sem=rsR_send.at[send_slot],
                recv_sem=rsR_recv.at[recv_slot],
                device_id=(right,),
                device_id_type=pl.DeviceIdType.MESH,
            )
            rdmaL = pltpu.make_async_remote_copy(
                src_ref=commL_ref.at[send_slot],
                dst_ref=commL_ref.at[recv_slot],
                send_sem=rsL_send.at[send_slot],
                recv_sem=rsL_recv.at[recv_slot],
                device_id=(left,),
                device_id_type=pl.DeviceIdType.MESH,
            )
            rdmaR.start()
            rdmaL.start()
            pc_ref[:, :HN] = pcR(cR(s + 1))
            pc_ref[:, HN:] = pcL(cL(s + 1))
            rdmaR.wait()
            sumR = (commR_ref[recv_slot].astype(jnp.float32)
                    + pc_ref[:, :HN])
            rdmaL.wait()
            sumL = (commL_ref[recv_slot].astype(jnp.float32)
                    + pc_ref[:, HN:])
            if s < N_DEV - 2:
                commR_ref[recv_slot] = sumR.astype(jnp.bfloat16)
                commL_ref[recv_slot] = sumL.astype(jnp.bfloat16)
            else:
                pc_ref[:, :HN] = jnp.maximum(sumR, 0.0)
                pc_ref[:, HN:] = jnp.maximum(sumL, 0.0)

        mineR = cR(N_DEV - 1)
        mineL = cL(N_DEV - 1)

        amax_ref[pl.ds(my, 1)] = (jnp.zeros((1, 8, 128), jnp.float32)
                                  + jnp.max(pc_ref[...]))
        bcasts = []
        for o in range(1, N_DEV):
            peer = lax.rem(my + o, N_DEV)
            r = pltpu.make_async_remote_copy(
                src_ref=amax_ref.at[my],
                dst_ref=amax_ref.at[my],
                send_sem=am_send.at[o],
                recv_sem=am_recv.at[my],
                device_id=(peer,),
                device_id_type=pl.DeviceIdType.MESH,
            )
            r.start()
            bcasts.append(r)
        for o in range(1, N_DEV):
            peer = lax.rem(my + o, N_DEV)
            wr = pltpu.make_async_remote_copy(
                src_ref=amax_ref.at[my],
                dst_ref=amax_ref.at[peer],
                send_sem=am_send.at[o],
                recv_sem=am_recv.at[peer],
                device_id=(peer,),
                device_id_type=pl.DeviceIdType.MESH,
            )
            wr.wait_recv()
        for r in bcasts:
            r.wait_send()

        gmax = jnp.max(amax_ref[...])

        scale = gmax / 127.0
        inv = jnp.where(gmax > 0.0, 127.0 / gmax, 0.0)
        qR_ref[0] = jnp.clip(jnp.round(pc_ref[:, :HN] * inv),
                             -127.0, 127.0).astype(jnp.int8)
        qL_ref[0] = jnp.clip(jnp.round(pc_ref[:, HN:] * inv),
                             -127.0, 127.0).astype(jnp.int8)
        out_ref[rows(mineR), :HN] = qR_ref[0].astype(jnp.float32) * scale
        out_ref[rows(mineL), HN:] = qL_ref[0].astype(jnp.float32) * scale

        for s in range(N_DEV - 1):
            send_slot, recv_slot = s % 2, (s + 1) % 2
            rdmaR = pltpu.make_async_remote_copy(
                src_ref=qR_ref.at[send_slot],
                dst_ref=qR_ref.at[recv_slot],
                send_sem=agR_send.at[send_slot],
                recv_sem=agR_recv.at[recv_slot],
                device_id=(right,),
                device_id_type=pl.DeviceIdType.MESH,
            )
            rdmaL = pltpu.make_async_remote_copy(
                src_ref=qL_ref.at[send_slot],
                dst_ref=qL_ref.at[recv_slot],
                send_sem=agL_send.at[send_slot],
                recv_sem=agL_recv.at[recv_slot],
                device_id=(left,),
                device_id_type=pl.DeviceIdType.MESH,
            )
            rdmaR.start()
            rdmaL.start()
            if s > 0:
                out_ref[rows(lax.rem(my - (s - 1) + 2 * N_DEV, N_DEV)), :HN] = (
                    qR_ref[send_slot].astype(jnp.float32) * scale)
                out_ref[rows(lax.rem(my + s - 1, N_DEV)), HN:] = (
                    qL_ref[send_slot].astype(jnp.float32) * scale)
            rdmaR.wait()
            rdmaL.wait()
        last = N_DEV - 2
        out_ref[rows(lax.rem(my - last + 2 * N_DEV, N_DEV)), :HN] = (
            qR_ref[(N_DEV - 1) % 2].astype(jnp.float32) * scale)
        out_ref[rows(lax.rem(my + last, N_DEV)), HN:] = (
            qL_ref[(N_DEV - 1) % 2].astype(jnp.float32) * scale)

    return pl.pallas_call(
        body,
        out_shape=jax.ShapeDtypeStruct((M, N), jnp.float32),
        in_specs=[pl.BlockSpec(memory_space=pltpu.VMEM),
                  pl.BlockSpec(memory_space=pltpu.VMEM)],
        out_specs=pl.BlockSpec(memory_space=pltpu.VMEM),
        scratch_shapes=[
            pltpu.VMEM((2, BLK, HN), jnp.bfloat16),
            pltpu.VMEM((2, BLK, HN), jnp.bfloat16),
            pltpu.VMEM((BLK, N), jnp.float32),
            pltpu.VMEM((N_DEV, 8, 128), jnp.float32),
            pltpu.VMEM((2, BLK, HN), jnp.int8),
            pltpu.VMEM((2, BLK, HN), jnp.int8),
            pltpu.SemaphoreType.DMA((2,)),
            pltpu.SemaphoreType.DMA((2,)),
            pltpu.SemaphoreType.DMA((2,)),
            pltpu.SemaphoreType.DMA((2,)),
            pltpu.SemaphoreType.DMA((N_DEV,)),
            pltpu.SemaphoreType.DMA((N_DEV,)),
            pltpu.SemaphoreType.DMA((2,)),
            pltpu.SemaphoreType.DMA((2,)),
            pltpu.SemaphoreType.DMA((2,)),
            pltpu.SemaphoreType.DMA((2,)),
        ],
        compiler_params=pltpu.CompilerParams(
            collective_id=0, vmem_limit_bytes=60 * 1024 * 1024),
    )(x, w_mat)


# device time: 185393 ns/iter; 3.8673x vs baseline; 1.0909x over previous
import jax
import jax.numpy as jnp
from jax import lax
from jax.experimental import pallas as pl
from jax.experimental.pallas import tpu as pltpu

N_DEV = 8
M, K_SHARD, N = 4096, 512, 2048
BLK = M // N_DEV
HN = N // 2
SUB = BLK // 2


def kernel(x, w_mat):
    def body(x_ref, w_ref, out_ref, commR_ref, commL_ref, pc_ref, amax_ref,
             qR_ref, qL_ref, rsR_send, rsR_recv, rsL_send, rsL_recv,
             am_send, am_recv, agR_send, agR_recv, agL_send, agL_recv):
        my = lax.axis_index("i")
        left = lax.rem(my + N_DEV - 1, N_DEV)
        right = lax.rem(my + 1, N_DEV)

        def rows(c):
            return pl.ds(c * BLK, BLK)

        def pcR(c):
            return jnp.dot(x_ref[rows(c), :], w_ref[:, :HN],
                           preferred_element_type=jnp.float32)

        def pcL(c):
            return jnp.dot(x_ref[rows(c), :], w_ref[:, HN:],
                           preferred_element_type=jnp.float32)

        def cR(s):
            return lax.rem(my - s + 2 * N_DEV, N_DEV)

        def cL(s):
            return lax.rem(my + s, N_DEV)

        def sub_rows(k):
            return pl.ds(k * SUB, SUB)

        commR_ref[0] = pcR(cR(0)).astype(jnp.bfloat16)
        commL_ref[0] = pcL(cL(0)).astype(jnp.bfloat16)

        barrier_sem = pltpu.get_barrier_semaphore()
        for nbr in [left, right]:
            pl.semaphore_signal(barrier_sem, inc=1, device_id=(nbr,),
                                device_id_type=pl.DeviceIdType.MESH)
        pl.semaphore_wait(barrier_sem, 2)

        def rs_desc(s, k, rightward):
            comm = commR_ref if rightward else commL_ref
            ssem = rsR_send if rightward else rsL_send
            rsem = rsR_recv if rightward else rsL_recv
            return pltpu.make_async_remote_copy(
                src_ref=comm.at[s % 2, sub_rows(k)],
                dst_ref=comm.at[(s + 1) % 2, sub_rows(k)],
                send_sem=ssem.at[s % 2, k],
                recv_sem=rsem.at[(s + 1) % 2, k],
                device_id=(right if rightward else left,),
                device_id_type=pl.DeviceIdType.MESH,
            )

        desc = {(s, k, rw): rs_desc(s, k, rw)
                for s in range(N_DEV - 1) for k in (0, 1)
                for rw in (True, False)}

        for k in (0, 1):
            desc[(0, k, True)].start()
            desc[(0, k, False)].start()
        pc_ref[:, :HN] = pcR(cR(1))
        pc_ref[:, HN:] = pcL(cL(1))

        for s in range(N_DEV - 1):
            final = s == N_DEV - 2
            recv_slot = (s + 1) % 2
            for k in (0, 1):
                for rw in (True, False):
                    comm = commR_ref if rw else commL_ref
                    cols = slice(0, HN) if rw else slice(HN, N)
                    d = desc[(s, k, rw)]
                    d.wait_recv()
                    if s >= 1:
                        desc[(s - 1, k, rw)].wait_send()
                    sub_sum = (comm[recv_slot, sub_rows(k)]
                               .astype(jnp.float32)
                               + pc_ref[sub_rows(k), cols])
                    if not final:
                        comm[recv_slot, sub_rows(k)] = (
                            sub_sum.astype(jnp.bfloat16))
                        desc[(s + 1, k, rw)].start()
                    else:
                        pc_ref[sub_rows(k), cols] = jnp.maximum(sub_sum, 0.0)
            if s < N_DEV - 2:
                pc_ref[:, :HN] = pcR(cR(s + 2))
                pc_ref[:, HN:] = pcL(cL(s + 2))
        for k in (0, 1):
            desc[(N_DEV - 2, k, True)].wait_send()
            desc[(N_DEV - 2, k, False)].wait_send()

        mineR = cR(N_DEV - 1)
        mineL = cL(N_DEV - 1)

        amax_ref[pl.ds(my, 1)] = (jnp.zeros((1, 8, 128), jnp.float32)
                                  + jnp.max(pc_ref[...]))
        bcasts = []
        for o in range(1, N_DEV):
            peer = lax.rem(my + o, N_DEV)
            r = pltpu.make_async_remote_copy(
                src_ref=amax_ref.at[my],
                dst_ref=amax_ref.at[my],
                send_sem=am_send.at[o],
                recv_sem=am_recv.at[my],
                device_id=(peer,),
                device_id_type=pl.DeviceIdType.MESH,
            )
            r.start()
            bcasts.append(r)
        for o in range(1, N_DEV):
            peer = lax.rem(my + o, N_DEV)
            wr = pltpu.make_async_remote_copy(
                src_ref=amax_ref.at[my],
                dst_ref=amax_ref.at[peer],
                send_sem=am_send.at[o],
                recv_sem=am_recv.at[peer],
                device_id=(peer,),
                device_id_type=pl.DeviceIdType.MESH,
            )
            wr.wait_recv()
        for r in bcasts:
            r.wait_send()

        gmax = jnp.max(amax_ref[...])

        scale = gmax / 127.0
        inv = jnp.where(gmax > 0.0, 127.0 / gmax, 0.0)
        qR_ref[0] = jnp.clip(jnp.round(pc_ref[:, :HN] * inv),
                             -127.0, 127.0).astype(jnp.int8)
        qL_ref[0] = jnp.clip(jnp.round(pc_ref[:, HN:] * inv),
                             -127.0, 127.0).astype(jnp.int8)
        out_ref[rows(mineR), :HN] = qR_ref[0].astype(jnp.float32) * scale
        out_ref[rows(mineL), HN:] = qL_ref[0].astype(jnp.float32) * scale

        for s in range(N_DEV - 1):
            send_slot, recv_slot = s % 2, (s + 1) % 2
            rdmaR = pltpu.make_async_remote_copy(
                src_ref=qR_ref.at[send_slot],
                dst_ref=qR_ref.at[recv_slot],
                send_sem=agR_send.at[send_slot],
                recv_sem=agR_recv.at[recv_slot],
                device_id=(right,),
                device_id_type=pl.DeviceIdType.MESH,
            )
            rdmaL = pltpu.make_async_remote_copy(
                src_ref=qL_ref.at[send_slot],
                dst_ref=qL_ref.at[recv_slot],
                send_sem=agL_send.at[send_slot],
                recv_sem=agL_recv.at[recv_slot],
                device_id=(left,),
                device_id_type=pl.DeviceIdType.MESH,
            )
            rdmaR.start()
            rdmaL.start()
            if s > 0:
                out_ref[rows(lax.rem(my - (s - 1) + 2 * N_DEV, N_DEV)), :HN] = (
                    qR_ref[send_slot].astype(jnp.float32) * scale)
                out_ref[rows(lax.rem(my + s - 1, N_DEV)), HN:] = (
                    qL_ref[send_slot].astype(jnp.float32) * scale)
            rdmaR.wait()
            rdmaL.wait()
        last = N_DEV - 2
        out_ref[rows(lax.rem(my - last + 2 * N_DEV, N_DEV)), :HN] = (
            qR_ref[(N_DEV - 1) % 2].astype(jnp.float32) * scale)
        out_ref[rows(lax.rem(my + last, N_DEV)), HN:] = (
            qL_ref[(N_DEV - 1) % 2].astype(jnp.float32) * scale)

    return pl.pallas_call(
        body,
        out_shape=jax.ShapeDtypeStruct((M, N), jnp.float32),
        in_specs=[pl.BlockSpec(memory_space=pltpu.VMEM),
                  pl.BlockSpec(memory_space=pltpu.VMEM)],
        out_specs=pl.BlockSpec(memory_space=pltpu.VMEM),
        scratch_shapes=[
            pltpu.VMEM((2, BLK, HN), jnp.bfloat16),
            pltpu.VMEM((2, BLK, HN), jnp.bfloat16),
            pltpu.VMEM((BLK, N), jnp.float32),
            pltpu.VMEM((N_DEV, 8, 128), jnp.float32),
            pltpu.VMEM((2, BLK, HN), jnp.int8),
            pltpu.VMEM((2, BLK, HN), jnp.int8),
            pltpu.SemaphoreType.DMA((2, 2)),
            pltpu.SemaphoreType.DMA((2, 2)),
            pltpu.SemaphoreType.DMA((2, 2)),
            pltpu.SemaphoreType.DMA((2, 2)),
            pltpu.SemaphoreType.DMA((N_DEV,)),
            pltpu.SemaphoreType.DMA((N_DEV,)),
            pltpu.SemaphoreType.DMA((2,)),
            pltpu.SemaphoreType.DMA((2,)),
            pltpu.SemaphoreType.DMA((2,)),
            pltpu.SemaphoreType.DMA((2,)),
        ],
        compiler_params=pltpu.CompilerParams(
            collective_id=0, vmem_limit_bytes=60 * 1024 * 1024),
    )(x, w_mat)


# device time: 170956 ns/iter; 4.1939x vs baseline; 1.0844x over previous
import jax
import jax.numpy as jnp
from jax import lax
from jax.experimental import pallas as pl
from jax.experimental.pallas import tpu as pltpu

N_DEV = 8
M, K_SHARD, N = 4096, 512, 2048
BLK = M // N_DEV
HN = N // 2
SUB = BLK // 2


def kernel(x, w_mat):
    def body(x_ref, w_ref, out_ref, commR_ref, commL_ref, pc_ref, amax_ref,
             qR_ref, qL_ref, rsR_send, rsR_recv, rsL_send, rsL_recv,
             am_send, am_recv, agR_send, agR_recv, agL_send, agL_recv):
        my = lax.axis_index("i")
        left = lax.rem(my + N_DEV - 1, N_DEV)
        right = lax.rem(my + 1, N_DEV)

        def rows(c):
            return pl.ds(c * BLK, BLK)

        def pcR(c):
            return jnp.dot(x_ref[rows(c), :], w_ref[:, :HN],
                           preferred_element_type=jnp.float32)

        def pcL(c):
            return jnp.dot(x_ref[rows(c), :], w_ref[:, HN:],
                           preferred_element_type=jnp.float32)

        def cR(s):
            return lax.rem(my - s + 2 * N_DEV, N_DEV)

        def cL(s):
            return lax.rem(my + s, N_DEV)

        def sub_rows(k):
            return pl.ds(k * SUB, SUB)

        commR_ref[0] = pcR(cR(0)).astype(jnp.bfloat16)
        commL_ref[0] = pcL(cL(0)).astype(jnp.bfloat16)

        barrier_sem = pltpu.get_barrier_semaphore()
        for nbr in [left, right]:
            pl.semaphore_signal(barrier_sem, inc=1, device_id=(nbr,),
                                device_id_type=pl.DeviceIdType.MESH)
        pl.semaphore_wait(barrier_sem, 2)

        def rs_desc(s, k, rightward):
            comm = commR_ref if rightward else commL_ref
            ssem = rsR_send if rightward else rsL_send
            rsem = rsR_recv if rightward else rsL_recv
            return pltpu.make_async_remote_copy(
                src_ref=comm.at[s % 2, sub_rows(k)],
                dst_ref=comm.at[(s + 1) % 2, sub_rows(k)],
                send_sem=ssem.at[s % 2, k],
                recv_sem=rsem.at[(s + 1) % 2, k],
                device_id=(right if rightward else left,),
                device_id_type=pl.DeviceIdType.MESH,
            )

        desc = {(s, k, rw): rs_desc(s, k, rw)
                for s in range(N_DEV - 1) for k in (0, 1)
                for rw in (True, False)}

        for k in (0, 1):
            desc[(0, k, True)].start()
            desc[(0, k, False)].start()
        pc_ref[:, :HN] = pcR(cR(1))
        pc_ref[:, HN:] = pcL(cL(1))

        for s in range(N_DEV - 1):
            final = s == N_DEV - 2
            recv_slot = (s + 1) % 2
            for k in (0, 1):
                for rw in (True, False):
                    comm = commR_ref if rw else commL_ref
                    cols = slice(0, HN) if rw else slice(HN, N)
                    d = desc[(s, k, rw)]
                    d.wait_recv()
                    if s >= 1:
                        desc[(s - 1, k, rw)].wait_send()
                    sub_sum = (comm[recv_slot, sub_rows(k)]
                               .astype(jnp.float32)
                               + pc_ref[sub_rows(k), cols])
                    if not final:
                        comm[recv_slot, sub_rows(k)] = (
                            sub_sum.astype(jnp.bfloat16))
                        desc[(s + 1, k, rw)].start()
                    else:
                        pc_ref[sub_rows(k), cols] = jnp.maximum(sub_sum, 0.0)
            if s < N_DEV - 2:
                pc_ref[:, :HN] = pcR(cR(s + 2))
                pc_ref[:, HN:] = pcL(cL(s + 2))
        for k in (0, 1):
            desc[(N_DEV - 2, k, True)].wait_send()
            desc[(N_DEV - 2, k, False)].wait_send()

        mineR = cR(N_DEV - 1)
        mineL = cL(N_DEV - 1)

        amax_ref[pl.ds(my, 1)] = (jnp.zeros((1, 8, 128), jnp.float32)
                                  + jnp.max(pc_ref[...]))
        bcasts = []
        for o in range(1, N_DEV):
            peer = lax.rem(my + o, N_DEV)
            r = pltpu.make_async_remote_copy(
                src_ref=amax_ref.at[my],
                dst_ref=amax_ref.at[my],
                send_sem=am_send.at[o],
                recv_sem=am_recv.at[my],
                device_id=(peer,),
                device_id_type=pl.DeviceIdType.MESH,
            )
            r.start()
            bcasts.append(r)
        for o in range(1, N_DEV):
            peer = lax.rem(my + o, N_DEV)
            wr = pltpu.make_async_remote_copy(
                src_ref=amax_ref.at[my],
                dst_ref=amax_ref.at[peer],
                send_sem=am_send.at[o],
                recv_sem=am_recv.at[peer],
                device_id=(peer,),
                device_id_type=pl.DeviceIdType.MESH,
            )
            wr.wait_recv()
        for r in bcasts:
            r.wait_send()

        gmax = jnp.max(amax_ref[...])

        scale = gmax / 127.0
        inv = jnp.where(gmax > 0.0, 127.0 / gmax, 0.0)
        qR_ref[0] = jnp.clip(jnp.round(pc_ref[:, :HN] * inv),
                             -127.0, 127.0).astype(jnp.int8)
        qL_ref[0] = jnp.clip(jnp.round(pc_ref[:, HN:] * inv),
                             -127.0, 127.0).astype(jnp.int8)

        def ag_desc(s, k, rightward):
            q = qR_ref if rightward else qL_ref
            ssem = agR_send if rightward else agL_send
            rsem = agR_recv if rightward else agL_recv
            return pltpu.make_async_remote_copy(
                src_ref=q.at[s % 2, sub_rows(k)],
                dst_ref=q.at[(s + 1) % 2, sub_rows(k)],
                send_sem=ssem.at[s % 2, k],
                recv_sem=rsem.at[(s + 1) % 2, k],
                device_id=(right if rightward else left,),
                device_id_type=pl.DeviceIdType.MESH,
            )

        agd = {(s, k, rw): ag_desc(s, k, rw)
               for s in range(N_DEV - 1) for k in (0, 1)
               for rw in (True, False)}

        for k in (0, 1):
            agd[(0, k, True)].start()
            agd[(0, k, False)].start()
        out_ref[rows(mineR), :HN] = qR_ref[0].astype(jnp.float32) * scale
        out_ref[rows(mineL), HN:] = qL_ref[0].astype(jnp.float32) * scale

        for s in range(N_DEV - 1):
            recv_slot = (s + 1) % 2
            for k in (0, 1):
                for rw in (True, False):
                    d = agd[(s, k, rw)]
                    d.wait_recv()
                    if s >= 1:
                        agd[(s - 1, k, rw)].wait_send()
                    if s < N_DEV - 2:
                        agd[(s + 1, k, rw)].start()
                    q = qR_ref if rw else qL_ref
                    c = (lax.rem(my - s + 2 * N_DEV, N_DEV) if rw
                         else lax.rem(my + s, N_DEV))
                    cols = slice(0, HN) if rw else slice(HN, N)
                    out_ref[pl.ds(c * BLK + k * SUB, SUB), cols] = (
                        q[recv_slot, sub_rows(k)].astype(jnp.float32) * scale)
        for k in (0, 1):
            agd[(N_DEV - 2, k, True)].wait_send()
            agd[(N_DEV - 2, k, False)].wait_send()

    return pl.pallas_call(
        body,
        out_shape=jax.ShapeDtypeStruct((M, N), jnp.float32),
        in_specs=[pl.BlockSpec(memory_space=pltpu.VMEM),
                  pl.BlockSpec(memory_space=pltpu.VMEM)],
        out_specs=pl.BlockSpec(memory_space=pltpu.VMEM),
        scratch_shapes=[
            pltpu.VMEM((2, BLK, HN), jnp.bfloat16),
            pltpu.VMEM((2, BLK, HN), jnp.bfloat16),
            pltpu.VMEM((BLK, N), jnp.float32),
            pltpu.VMEM((N_DEV, 8, 128), jnp.float32),
            pltpu.VMEM((2, BLK, HN), jnp.int8),
            pltpu.VMEM((2, BLK, HN), jnp.int8),
            pltpu.SemaphoreType.DMA((2, 2)),
            pltpu.SemaphoreType.DMA((2, 2)),
            pltpu.SemaphoreType.DMA((2, 2)),
            pltpu.SemaphoreType.DMA((2, 2)),
            pltpu.SemaphoreType.DMA((N_DEV,)),
            pltpu.SemaphoreType.DMA((N_DEV,)),
            pltpu.SemaphoreType.DMA((2, 2)),
            pltpu.SemaphoreType.DMA((2, 2)),
            pltpu.SemaphoreType.DMA((2, 2)),
            pltpu.SemaphoreType.DMA((2, 2)),
        ],
        compiler_params=pltpu.CompilerParams(
            collective_id=0, vmem_limit_bytes=60 * 1024 * 1024),
    )(x, w_mat)


# device time: 169797 ns/iter; 4.2225x vs baseline; 1.0068x over previous
import jax
import jax.numpy as jnp
from jax import lax
from jax.experimental import pallas as pl
from jax.experimental.pallas import tpu as pltpu

N_DEV = 8
M, K_SHARD, N = 4096, 512, 2048
BLK = M // N_DEV
HN = N // 2
SUB = BLK // 2


def kernel(x, w_mat):
    def body(x_ref, w_ref, out_ref, commR_ref, commL_ref, pc_ref, amax_ref,
             qR_ref, qL_ref, rsR_send, rsR_recv, rsL_send, rsL_recv,
             am_send, am_recv, agR_send, agR_recv, agL_send, agL_recv):
        my = lax.axis_index("i")

        def perm(v):
            return jnp.where(v < 4, v, 11 - v)

        my_pos = perm(my)
        right = perm(lax.rem(my_pos + 1, N_DEV))
        left = perm(lax.rem(my_pos + N_DEV - 1, N_DEV))

        def rows(c):
            return pl.ds(c * BLK, BLK)

        def pcR(c):
            return jnp.dot(x_ref[rows(c), :], w_ref[:, :HN],
                           preferred_element_type=jnp.float32)

        def pcL(c):
            return jnp.dot(x_ref[rows(c), :], w_ref[:, HN:],
                           preferred_element_type=jnp.float32)

        def cR(s):
            return lax.rem(my_pos - s + 2 * N_DEV, N_DEV)

        def cL(s):
            return lax.rem(my_pos + s, N_DEV)

        def sub_rows(k):
            return pl.ds(k * SUB, SUB)

        def pcR_sub(c, k):
            return jnp.dot(x_ref[pl.ds(c * BLK + k * SUB, SUB), :],
                           w_ref[:, :HN], preferred_element_type=jnp.float32)

        def pcL_sub(c, k):
            return jnp.dot(x_ref[pl.ds(c * BLK + k * SUB, SUB), :],
                           w_ref[:, HN:], preferred_element_type=jnp.float32)

        commR_ref[0, sub_rows(0)] = pcR_sub(cR(0), 0).astype(jnp.bfloat16)
        commL_ref[0, sub_rows(0)] = pcL_sub(cL(0), 0).astype(jnp.bfloat16)

        barrier_sem = pltpu.get_barrier_semaphore()
        for nbr in [left, right]:
            pl.semaphore_signal(barrier_sem, inc=1, device_id=(nbr,),
                                device_id_type=pl.DeviceIdType.MESH)
        pl.semaphore_wait(barrier_sem, 2)

        def rs_desc(s, k, rightward):
            comm = commR_ref if rightward else commL_ref
            ssem = rsR_send if rightward else rsL_send
            rsem = rsR_recv if rightward else rsL_recv
            return pltpu.make_async_remote_copy(
                src_ref=comm.at[s % 2, sub_rows(k)],
                dst_ref=comm.at[(s + 1) % 2, sub_rows(k)],
                send_sem=ssem.at[s % 2, k],
                recv_sem=rsem.at[(s + 1) % 2, k],
                device_id=(right if rightward else left,),
                device_id_type=pl.DeviceIdType.MESH,
            )

        desc = {(s, k, rw): rs_desc(s, k, rw)
                for s in range(N_DEV - 1) for k in (0, 1)
                for rw in (True, False)}

        desc[(0, 0, True)].start()
        desc[(0, 0, False)].start()
        commR_ref[0, sub_rows(1)] = pcR_sub(cR(0), 1).astype(jnp.bfloat16)
        commL_ref[0, sub_rows(1)] = pcL_sub(cL(0), 1).astype(jnp.bfloat16)
        desc[(0, 1, True)].start()
        desc[(0, 1, False)].start()
        pc_ref[:, :HN] = pcR(cR(1))
        pc_ref[:, HN:] = pcL(cL(1))

        for s in range(N_DEV - 1):
            final = s == N_DEV - 2
            recv_slot = (s + 1) % 2
            for k in (0, 1):
                for rw in (True, False):
                    comm = commR_ref if rw else commL_ref
                    cols = slice(0, HN) if rw else slice(HN, N)
                    d = desc[(s, k, rw)]
                    d.wait_recv()
                    if s >= 1:
                        desc[(s - 1, k, rw)].wait_send()
                    sub_sum = (comm[recv_slot, sub_rows(k)]
                               .astype(jnp.float32)
                               + pc_ref[sub_rows(k), cols])
                    if not final:
                        comm[recv_slot, sub_rows(k)] = (
                            sub_sum.astype(jnp.bfloat16))
                        desc[(s + 1, k, rw)].start()
                    else:
                        pc_ref[sub_rows(k), cols] = jnp.maximum(sub_sum, 0.0)
            if s < N_DEV - 2:
                pc_ref[:, :HN] = pcR(cR(s + 2))
                pc_ref[:, HN:] = pcL(cL(s + 2))
        for k in (0, 1):
            desc[(N_DEV - 2, k, True)].wait_send()
            desc[(N_DEV - 2, k, False)].wait_send()

        mineR = cR(N_DEV - 1)
        mineL = cL(N_DEV - 1)

        amax_ref[pl.ds(my, 1)] = (jnp.zeros((1, 8, 128), jnp.float32)
                                  + jnp.max(pc_ref[...]))
        bcasts = []
        for o in range(1, N_DEV):
            peer = lax.rem(my + o, N_DEV)
            r = pltpu.make_async_remote_copy(
                src_ref=amax_ref.at[my],
                dst_ref=amax_ref.at[my],
                send_sem=am_send.at[o],
                recv_sem=am_recv.at[my],
                device_id=(peer,),
                device_id_type=pl.DeviceIdType.MESH,
            )
            r.start()
            bcasts.append(r)
        for o in range(1, N_DEV):
            peer = lax.rem(my + o, N_DEV)
            wr = pltpu.make_async_remote_copy(
                src_ref=amax_ref.at[my],
                dst_ref=amax_ref.at[peer],
                send_sem=am_send.at[o],
                recv_sem=am_recv.at[peer],
                device_id=(peer,),
                device_id_type=pl.DeviceIdType.MESH,
            )
            wr.wait_recv()
        for r in bcasts:
            r.wait_send()

        gmax = jnp.max(amax_ref[...])

        scale = gmax / 127.0
        inv = jnp.where(gmax > 0.0, 127.0 / gmax, 0.0)
        qR_ref[0] = jnp.clip(jnp.round(pc_ref[:, :HN] * inv),
                             -127.0, 127.0).astype(jnp.int8)
        qL_ref[0] = jnp.clip(jnp.round(pc_ref[:, HN:] * inv),
                             -127.0, 127.0).astype(jnp.int8)

        def ag_desc(s, k, rightward):
            q = qR_ref if rightward else qL_ref
            ssem = agR_send if rightward else agL_send
            rsem = agR_recv if rightward else agL_recv
            return pltpu.make_async_remote_copy(
                src_ref=q.at[s % 2, sub_rows(k)],
                dst_ref=q.at[(s + 1) % 2, sub_rows(k)],
                send_sem=ssem.at[s % 2, k],
                recv_sem=rsem.at[(s + 1) % 2, k],
                device_id=(right if rightward else left,),
                device_id_type=pl.DeviceIdType.MESH,
            )

        agd = {(s, k, rw): ag_desc(s, k, rw)
               for s in range(N_DEV - 1) for k in (0, 1)
               for rw in (True, False)}

        for k in (0, 1):
            agd[(0, k, True)].start()
            agd[(0, k, False)].start()
        out_ref[rows(mineR), :HN] = qR_ref[0].astype(jnp.float32) * scale
        out_ref[rows(mineL), HN:] = qL_ref[0].astype(jnp.float32) * scale

        for s in range(N_DEV - 1):
            recv_slot = (s + 1) % 2
            for k in (0, 1):
                for rw in (True, False):
                    d = agd[(s, k, rw)]
                    d.wait_recv()
                    if s >= 1:
                        agd[(s - 1, k, rw)].wait_send()
                    if s < N_DEV - 2:
                        agd[(s + 1, k, rw)].start()
                    q = qR_ref if rw else qL_ref
                    c = (lax.rem(my_pos - s + 2 * N_DEV, N_DEV) if rw
                         else lax.rem(my_pos + s, N_DEV))
                    cols = slice(0, HN) if rw else slice(HN, N)
                    out_ref[pl.ds(c * BLK + k * SUB, SUB), cols] = (
                        q[recv_slot, sub_rows(k)].astype(jnp.float32) * scale)
        for k in (0, 1):
            agd[(N_DEV - 2, k, True)].wait_send()
            agd[(N_DEV - 2, k, False)].wait_send()

    return pl.pallas_call(
        body,
        out_shape=jax.ShapeDtypeStruct((M, N), jnp.float32),
        in_specs=[pl.BlockSpec(memory_space=pltpu.VMEM),
                  pl.BlockSpec(memory_space=pltpu.VMEM)],
        out_specs=pl.BlockSpec(memory_space=pltpu.VMEM),
        scratch_shapes=[
            pltpu.VMEM((2, BLK, HN), jnp.bfloat16),
            pltpu.VMEM((2, BLK, HN), jnp.bfloat16),
            pltpu.VMEM((BLK, N), jnp.float32),
            pltpu.VMEM((N_DEV, 8, 128), jnp.float32),
            pltpu.VMEM((2, BLK, HN), jnp.int8),
            pltpu.VMEM((2, BLK, HN), jnp.int8),
            pltpu.SemaphoreType.DMA((2, 2)),
            pltpu.SemaphoreType.DMA((2, 2)),
            pltpu.SemaphoreType.DMA((2, 2)),
            pltpu.SemaphoreType.DMA((2, 2)),
            pltpu.SemaphoreType.DMA((N_DEV,)),
            pltpu.SemaphoreType.DMA((N_DEV,)),
            pltpu.SemaphoreType.DMA((2, 2)),
            pltpu.SemaphoreType.DMA((2, 2)),
            pltpu.SemaphoreType.DMA((2, 2)),
            pltpu.SemaphoreType.DMA((2, 2)),
        ],
        compiler_params=pltpu.CompilerParams(
            collective_id=0, vmem_limit_bytes=60 * 1024 * 1024),
    )(x, w_mat)
